# baseline (speedup 1.0000x reference)
import jax
import jax.numpy as jnp
from jax.experimental import pallas as pl
from jax.experimental.pallas import tpu as pltpu

_BB = 8
_NSEM = 8


def _broadcast_dma_body(w_ref, o_ref, buf_ref, *sems):
    buf_ref[...] = jnp.broadcast_to(w_ref[...][None], buf_ref.shape)
    n = o_ref.shape[0] // _BB
    copies = [
        pltpu.make_async_copy(
            buf_ref, o_ref.at[pl.ds(i * _BB, _BB)], sems[i % _NSEM]
        )
        for i in range(n)
    ]
    for c in copies:
        c.start()
    for c in copies:
        c.wait()


def kernel(x, W):
    B, P, D = x.shape
    out = pl.pallas_call(
        _broadcast_dma_body,
        in_specs=[pl.BlockSpec(memory_space=pltpu.MemorySpace.VMEM)],
        out_specs=pl.BlockSpec(memory_space=pltpu.MemorySpace.HBM),
        out_shape=jax.ShapeDtypeStruct((B, P, D), W.dtype),
        scratch_shapes=[pltpu.VMEM((_BB, P, D), W.dtype)]
        + [pltpu.SemaphoreType.DMA] * _NSEM,
    )(W)
    return out


# speedup vs baseline: 4.8474x; 4.8474x over previous
import jax
import jax.numpy as jnp
from jax.experimental import pallas as pl


def _broadcast_body(w_ref, o_ref):
    o_ref[...] = jnp.broadcast_to(w_ref[...][None], o_ref.shape)


def kernel(x, W):
    B, P, D = x.shape
    BB = 8
    Wt = jnp.swapaxes(W, 0, 1)
    out_t = pl.pallas_call(
        _broadcast_body,
        grid=(B // BB,),
        in_specs=[pl.BlockSpec((D, P), lambda i: (0, 0))],
        out_specs=pl.BlockSpec((BB, D, P), lambda i: (i, 0, 0)),
        out_shape=jax.ShapeDtypeStruct((B, D, P), W.dtype),
    )(Wt)
    return jnp.swapaxes(out_t, 1, 2)
